# baseline (speedup 1.0000x reference)
import jax
import jax.numpy as jnp
from jax.experimental import pallas as pl
from jax.experimental.pallas import tpu as pltpu

MU = 0.01
LANES = 128


def _qnet_kernel(obs_ref, act_ref, w1o_ref, w1a_ref, b1_ref, rx_ref, rte_ref,
                 ewa_ref, eba_ref, ewb_hbm, ebb_ref, w6_ref, b6_ref, out_ref,
                 eblk_ref, sem):
    f32 = jnp.float32
    bf16 = jnp.bfloat16
    T = rte_ref.shape[0]
    H2 = w1o_ref.shape[1]
    H = H2 // 2
    E2 = rx_ref.shape[1]
    E = E2 // 2
    EH = E * H
    OB = obs_ref.shape[1] - T
    TB = obs_ref.shape[0]
    npair = E // 2

    copies = [
        pltpu.make_async_copy(
            ewb_hbm.at[pl.ds(2 * H * p, 2 * H), pl.ds(2 * H * p, 2 * H)],
            eblk_ref.at[pl.ds(2 * H * p, 2 * H), :], sem)
        for p in range(2 * npair)
    ]
    for c in copies:
        c.start()

    base = obs_ref[:, :OB].astype(bf16)
    onehot = obs_ref[:, OB:].astype(bf16)

    x1 = (jnp.dot(base, w1o_ref[...].astype(bf16), preferred_element_type=f32)
          + jnp.dot(act_ref[...].astype(bf16), w1a_ref[...].astype(bf16),
                    preferred_element_type=f32)
          + b1_ref[...])
    x1 = jnp.maximum(x1, 0.0)
    x1b = x1.astype(bf16)

    logits = (jnp.dot(x1b, rx_ref[...].astype(bf16),
                      preferred_element_type=f32)
              + jnp.dot(onehot, rte_ref[...].astype(bf16),
                        preferred_element_type=f32))
    grp = jax.lax.broadcasted_iota(jnp.int32, logits.shape, 1) >= E
    neg = jnp.float32(-jnp.inf)
    m1 = jnp.max(jnp.where(grp, neg, logits), axis=-1, keepdims=True)
    m2 = jnp.max(jnp.where(grp, logits, neg), axis=-1, keepdims=True)
    e = jnp.exp(logits - jnp.where(grp, m2, m1))
    s1 = jnp.sum(jnp.where(grp, 0.0, e), axis=-1, keepdims=True)
    s2 = jnp.sum(jnp.where(grp, e, 0.0), axis=-1, keepdims=True)
    ew = e / jnp.where(grp, s2, s1)

    h1 = jnp.maximum(
        jnp.dot(x1b[:, :H], ewa_ref[:H, :EH].astype(bf16),
                preferred_element_type=f32).astype(bf16)
        + eba_ref[:, :EH].astype(bf16), bf16(0.0))
    h2 = jnp.maximum(
        jnp.dot(x1b[:, H:], ewa_ref[H:, EH:].astype(bf16),
                preferred_element_type=f32).astype(bf16)
        + eba_ref[:, EH:].astype(bf16), bf16(0.0))

    rows = jax.lax.broadcasted_iota(jnp.int32, (2 * EH, LANES), 0)
    cols = jax.lax.broadcasted_iota(jnp.int32, (2 * EH, LANES), 1)
    v = w6_ref[:, 0:1] + w6_ref[:, 1:2]
    w6blk = jnp.where(cols == rows // H, v, 0.0).astype(bf16)

    for c in copies:
        c.wait()

    S = jnp.zeros((TB, LANES), f32)
    for p in range(npair):
        o = 2 * H * p
        g = jnp.maximum(
            jnp.dot(h1[:, o:o + 2 * H], eblk_ref[o:o + 2 * H, :].astype(bf16),
                    preferred_element_type=f32).astype(bf16)
            + ebb_ref[:, o:o + 2 * H].astype(bf16), bf16(0.0))
        S = S + jnp.dot(g, w6blk[o:o + 2 * H, :],
                        preferred_element_type=f32)
    for p in range(npair):
        o = 2 * H * p
        g = jnp.maximum(
            jnp.dot(h2[:, o:o + 2 * H],
                    eblk_ref[EH + o:EH + o + 2 * H, :].astype(bf16),
                    preferred_element_type=f32).astype(bf16)
            + ebb_ref[:, EH + o:EH + o + 2 * H].astype(bf16), bf16(0.0))
        S = S + jnp.dot(g, w6blk[EH + o:EH + o + 2 * H, :],
                        preferred_element_type=f32)

    prod = ew * S[:, :E2]
    lane = jax.lax.broadcasted_iota(jnp.int32, prod.shape, 1)
    q1 = jnp.sum(jnp.where(lane < E, prod, 0.0), axis=-1, keepdims=True)
    q2 = jnp.sum(jnp.where(lane >= E, prod, 0.0), axis=-1, keepdims=True)

    reg = (-(1.0 / E) * MU
           * jnp.sum(jnp.log(ew + 1e-6), axis=-1, keepdims=True))

    col = jax.lax.broadcasted_iota(jnp.int32, out_ref.shape, 1)
    q12 = (jnp.where(col == 0, q1, jnp.where(col == 1, q2, 0.0))
           + b6_ref[:, :out_ref.shape[1]])
    out_ref[...] = jnp.where(col == 2, reg, q12)


def _pick_tile(B, cap=4096):
    if B <= cap:
        return B
    for tb in range(cap, 7, -8):
        if B % tb == 0:
            return tb
    return B


def kernel(obs, action, w1o, w1a, b1, rx, rte, ewa, eba, ewb, ebb,
           rexp, w6pack, b6pack):
    B = obs.shape[0]
    OBT = obs.shape[1]
    A = action.shape[1]
    T = rte.shape[0]
    H2 = w1o.shape[1]
    E2 = rx.shape[1]
    EH2 = ewa.shape[1]
    H = H2 // 2
    EH = EH2 // 2

    TB = _pick_tile(B)
    grid = (B // TB,)
    row = lambda i: (i, 0)
    rep = lambda i: (0, 0)

    flops = 2 * B * (OBT * H2 + A * H2 + H2 * E2 + T * E2
                     + H * EH2 + 2 * H * EH2 + EH2 * LANES)
    bytes_accessed = 4 * (B * (OBT + A + LANES)
                          + OBT * H2 + A * H2 + H2 + H2 * E2 + T * E2
                          + H2 * EH2 + EH2 + 2 * EH2 * H2 + EH2
                          + EH2 * LANES + LANES)

    OUTW = 8
    out = pl.pallas_call(
        _qnet_kernel,
        out_shape=jax.ShapeDtypeStruct((B, OUTW), jnp.float32),
        grid=grid,
        in_specs=[
            pl.BlockSpec((TB, OBT), row),
            pl.BlockSpec((TB, A), row),
            pl.BlockSpec((OBT - T, H2), rep),
            pl.BlockSpec((A, H2), rep),
            pl.BlockSpec((1, H2), rep),
            pl.BlockSpec((H2, E2), rep),
            pl.BlockSpec((T, E2), rep),
            pl.BlockSpec((H2, EH2), rep),
            pl.BlockSpec((1, EH2), rep),
            pl.BlockSpec(memory_space=pl.ANY),
            pl.BlockSpec((1, EH2), rep),
            pl.BlockSpec((EH2, LANES), rep),
            pl.BlockSpec((1, LANES), rep),
        ],
        out_specs=pl.BlockSpec((TB, OUTW), row),
        scratch_shapes=[
            pltpu.VMEM((EH2, H2), jnp.float32),
            pltpu.SemaphoreType.DMA,
        ],
        compiler_params=pltpu.CompilerParams(
            dimension_semantics=("parallel",)),
        cost_estimate=pl.CostEstimate(
            flops=flops, transcendentals=B * (2 * E2 + 2),
            bytes_accessed=bytes_accessed),
    )(obs, action, w1o, w1a, b1, rx, rte, ewa, eba, ewb, ebb,
      w6pack, b6pack)

    return out[:, 0:1], out[:, 1:2], out[:, 2]


# speedup vs baseline: 1.0135x; 1.0135x over previous
import jax
import jax.numpy as jnp
from jax.experimental import pallas as pl
from jax.experimental.pallas import tpu as pltpu

MU = 0.01
LANES = 128


def _qnet_kernel(obs_ref, act_ref, w1o_ref, w1a_ref, b1_ref, rx_ref, rte_ref,
                 ewa_ref, eba_ref, ewb_hbm, ebb_ref, w6_ref, b6_ref, out_ref,
                 eblk_ref, sem):
    f32 = jnp.float32
    bf16 = jnp.bfloat16
    T = rte_ref.shape[0]
    H2 = w1o_ref.shape[1]
    H = H2 // 2
    E2 = rx_ref.shape[1]
    E = E2 // 2
    EH = E * H
    OB = obs_ref.shape[1] - T
    TB = obs_ref.shape[0]
    npair = E // 2

    copies = [
        pltpu.make_async_copy(
            ewb_hbm.at[pl.ds(2 * H * p, 2 * H), pl.ds(2 * H * p, 2 * H)],
            eblk_ref.at[pl.ds(2 * H * p, 2 * H), :], sem)
        for p in range(2 * npair)
    ]
    for c in copies:
        c.start()

    base = obs_ref[:, :OB].astype(bf16)
    onehot = obs_ref[:, OB:].astype(bf16)

    x1 = (jnp.dot(base, w1o_ref[...].astype(bf16), preferred_element_type=f32)
          + jnp.dot(act_ref[...].astype(bf16), w1a_ref[...].astype(bf16),
                    preferred_element_type=f32)
          + b1_ref[...])
    x1 = jnp.maximum(x1, 0.0)
    x1b = x1.astype(bf16)

    logits = (jnp.dot(x1b, rx_ref[...].astype(bf16),
                      preferred_element_type=f32)
              + jnp.dot(onehot, rte_ref[...].astype(bf16),
                        preferred_element_type=f32))
    grp = jax.lax.broadcasted_iota(jnp.int32, logits.shape, 1) >= E
    neg = jnp.float32(-jnp.inf)
    m1 = jnp.max(jnp.where(grp, neg, logits), axis=-1, keepdims=True)
    m2 = jnp.max(jnp.where(grp, logits, neg), axis=-1, keepdims=True)
    e = jnp.exp(logits - jnp.where(grp, m2, m1))
    s1 = jnp.sum(jnp.where(grp, 0.0, e), axis=-1, keepdims=True)
    s2 = jnp.sum(jnp.where(grp, e, 0.0), axis=-1, keepdims=True)
    ew = e / jnp.where(grp, s2, s1)

    h1 = jnp.maximum(
        jnp.dot(x1b[:, :H], ewa_ref[:H, :EH].astype(bf16),
                preferred_element_type=f32).astype(bf16)
        + eba_ref[:, :EH].astype(bf16), bf16(0.0))
    h2 = jnp.maximum(
        jnp.dot(x1b[:, H:], ewa_ref[H:, EH:].astype(bf16),
                preferred_element_type=f32).astype(bf16)
        + eba_ref[:, EH:].astype(bf16), bf16(0.0))

    rows = jax.lax.broadcasted_iota(jnp.int32, (2 * EH, LANES), 0)
    cols = jax.lax.broadcasted_iota(jnp.int32, (2 * EH, LANES), 1)
    v = w6_ref[:, 0:1] + w6_ref[:, 1:2]
    w6blk = jnp.where(cols == rows // H, v, 0.0).astype(bf16)

    for c in copies:
        c.wait()

    S = jnp.zeros((TB, LANES), f32)
    for p in range(npair):
        o = 2 * H * p
        g = jnp.maximum(
            jnp.dot(h1[:, o:o + 2 * H], eblk_ref[o:o + 2 * H, :].astype(bf16),
                    preferred_element_type=f32).astype(bf16)
            + ebb_ref[:, o:o + 2 * H].astype(bf16), bf16(0.0))
        S = S + jnp.dot(g, w6blk[o:o + 2 * H, :],
                        preferred_element_type=f32)
    for p in range(npair):
        o = 2 * H * p
        g = jnp.maximum(
            jnp.dot(h2[:, o:o + 2 * H],
                    eblk_ref[EH + o:EH + o + 2 * H, :].astype(bf16),
                    preferred_element_type=f32).astype(bf16)
            + ebb_ref[:, EH + o:EH + o + 2 * H].astype(bf16), bf16(0.0))
        S = S + jnp.dot(g, w6blk[EH + o:EH + o + 2 * H, :],
                        preferred_element_type=f32)

    prod = ew * S[:, :E2]
    lane = jax.lax.broadcasted_iota(jnp.int32, prod.shape, 1)
    q1 = jnp.sum(jnp.where(lane < E, prod, 0.0), axis=-1, keepdims=True)
    q2 = jnp.sum(jnp.where(lane >= E, prod, 0.0), axis=-1, keepdims=True)

    reg = (-(1.0 / E) * MU
           * jnp.sum(jnp.log(ew + 1e-6), axis=-1, keepdims=True))

    col = jax.lax.broadcasted_iota(jnp.int32, out_ref.shape, 1)
    q12 = (jnp.where(col == 0, q1, jnp.where(col == 1, q2, 0.0))
           + b6_ref[:, :out_ref.shape[1]])
    out_ref[...] = jnp.where(col == 2, reg, q12)


def _pick_tile(B, cap=2048):
    if B <= cap:
        return B
    for tb in range(cap, 7, -8):
        if B % tb == 0:
            return tb
    return B


def kernel(obs, action, w1o, w1a, b1, rx, rte, ewa, eba, ewb, ebb,
           rexp, w6pack, b6pack):
    B = obs.shape[0]
    OBT = obs.shape[1]
    A = action.shape[1]
    T = rte.shape[0]
    H2 = w1o.shape[1]
    E2 = rx.shape[1]
    EH2 = ewa.shape[1]
    H = H2 // 2
    EH = EH2 // 2

    TB = _pick_tile(B)
    grid = (B // TB,)
    row = lambda i: (i, 0)
    rep = lambda i: (0, 0)

    flops = 2 * B * (OBT * H2 + A * H2 + H2 * E2 + T * E2
                     + H * EH2 + 2 * H * EH2 + EH2 * LANES)
    bytes_accessed = 4 * (B * (OBT + A + LANES)
                          + OBT * H2 + A * H2 + H2 + H2 * E2 + T * E2
                          + H2 * EH2 + EH2 + 2 * EH2 * H2 + EH2
                          + EH2 * LANES + LANES)

    OUTW = 8
    out = pl.pallas_call(
        _qnet_kernel,
        out_shape=jax.ShapeDtypeStruct((B, OUTW), jnp.float32),
        grid=grid,
        in_specs=[
            pl.BlockSpec((TB, OBT), row),
            pl.BlockSpec((TB, A), row),
            pl.BlockSpec((OBT - T, H2), rep),
            pl.BlockSpec((A, H2), rep),
            pl.BlockSpec((1, H2), rep),
            pl.BlockSpec((H2, E2), rep),
            pl.BlockSpec((T, E2), rep),
            pl.BlockSpec((H2, EH2), rep),
            pl.BlockSpec((1, EH2), rep),
            pl.BlockSpec(memory_space=pl.ANY),
            pl.BlockSpec((1, EH2), rep),
            pl.BlockSpec((EH2, LANES), rep),
            pl.BlockSpec((1, LANES), rep),
        ],
        out_specs=pl.BlockSpec((TB, OUTW), row),
        scratch_shapes=[
            pltpu.VMEM((EH2, H2), jnp.float32),
            pltpu.SemaphoreType.DMA,
        ],
        compiler_params=pltpu.CompilerParams(
            dimension_semantics=("parallel",)),
        cost_estimate=pl.CostEstimate(
            flops=flops, transcendentals=B * (2 * E2 + 2),
            bytes_accessed=bytes_accessed),
    )(obs, action, w1o, w1a, b1, rx, rte, ewa, eba, ewb, ebb,
      w6pack, b6pack)

    return out[:, 0:1], out[:, 1:2], out[:, 2]
